# baseline (speedup 1.0000x reference)
import functools

import jax
import jax.numpy as jnp
from jax import lax
from jax.experimental import pallas as pl
from jax.experimental.pallas import tpu as pltpu
from jax.experimental.pallas import tpu_sc as plsc

N = 256
D_IN = 2048
D_OUT = 1024
K = 256

NC, NS, L = 2, 16, 16
NW = NC * NS
R = D_OUT // NW

_mesh = plsc.VectorSubcoreMesh(core_axis_name="c", subcore_axis_name="s")


@functools.partial(
    pl.kernel,
    out_type=jax.ShapeDtypeStruct((D_OUT * D_IN,), jnp.float32),
    mesh=_mesh,
    scratch_types=[
        pltpu.VMEM((R, K), jnp.int32),
        pltpu.VMEM((R, K), jnp.float32),
        pltpu.VMEM((R * D_IN,), jnp.float32),
        pltpu.SemaphoreType.DMA,
        pltpu.SemaphoreType.DMA,
    ],
    compiler_params=pltpu.CompilerParams(needs_layout_passes=False),
)
def _densify(mask_hbm, w_hbm, wt_hbm, mask_v, w_v, rows_v, sem_in, sem_out):
    wid = lax.axis_index("s") * NC + lax.axis_index("c")
    base = wid * R
    cp_m = pltpu.async_copy(mask_hbm.at[pl.ds(base, R)], mask_v, sem_in)
    cp_w = pltpu.async_copy(w_hbm.at[pl.ds(base, R)], w_v, sem_in)

    zeros = jnp.zeros((L,), jnp.float32)

    @pl.loop(0, R * D_IN // L, unroll=32)
    def _zero(i):
        rows_v[pl.ds(i * L, L)] = zeros

    cp_m.wait()
    cp_w.wait()

    CH = 2
    RC = R // CH
    cps = []
    for c in range(CH):
        @plsc.parallel_loop(c * RC, (c + 1) * RC)
        def _scatter(r):
            off = (r * D_IN).astype(jnp.int32)
            for j in range(K // L):
                idx = mask_v[r, pl.ds(j * L, L)] + off
                val = w_v[r, pl.ds(j * L, L)]
                plsc.addupdate_scatter(rows_v, [idx], val)

        cps.append(pltpu.async_copy(
            rows_v.at[pl.ds(c * RC * D_IN, RC * D_IN)],
            wt_hbm.at[pl.ds((base + c * RC) * D_IN, RC * D_IN)],
            sem_out,
        ))
    for cp in cps:
        cp.wait()


BO = 512


def _mm_body(x_ref, wt_ref, b_ref, o_ref):
    wtb = wt_ref[...].reshape(BO, D_IN)
    o_ref[...] = (
        lax.dot_general(
            x_ref[...],
            wtb,
            dimension_numbers=(((1,), (1,)), ((), ())),
            preferred_element_type=jnp.float32,
            precision=lax.Precision.DEFAULT,
        )
        + b_ref[...]
    )


def _matmul(x, wt_flat, bias2d):
    return pl.pallas_call(
        _mm_body,
        grid=(D_OUT // BO,),
        in_specs=[
            pl.BlockSpec((N, D_IN), lambda i: (0, 0)),
            pl.BlockSpec((BO * D_IN,), lambda i: (i,)),
            pl.BlockSpec((1, BO), lambda i: (0, i)),
        ],
        out_specs=pl.BlockSpec((N, BO), lambda i: (0, i)),
        out_shape=jax.ShapeDtypeStruct((N, D_OUT), jnp.float32),
    )(x, wt_flat, bias2d)


def kernel(input, input_mask, condensed_weight, bias):
    wt = _densify(input_mask, condensed_weight)
    return _matmul(input, wt, bias.reshape(1, D_OUT))


# speedup vs baseline: 1.0147x; 1.0147x over previous
import functools

import jax
import jax.numpy as jnp
from jax import lax
from jax.experimental import pallas as pl
from jax.experimental.pallas import tpu as pltpu
from jax.experimental.pallas import tpu_sc as plsc

N = 256
D_IN = 2048
D_OUT = 1024
K = 256

NC, NS, L = 2, 16, 16
NW = NC * NS
R = D_OUT // NW

_mesh = plsc.VectorSubcoreMesh(core_axis_name="c", subcore_axis_name="s")


@functools.partial(
    pl.kernel,
    out_type=jax.ShapeDtypeStruct((D_OUT * D_IN,), jnp.float32),
    mesh=_mesh,
    scratch_types=[
        pltpu.VMEM((R, K), jnp.int32),
        pltpu.VMEM((R, K), jnp.float32),
        pltpu.VMEM((R * D_IN,), jnp.float32),
        pltpu.SemaphoreType.DMA,
        pltpu.SemaphoreType.DMA,
    ],
    compiler_params=pltpu.CompilerParams(needs_layout_passes=False),
)
def _densify(mask_hbm, w_hbm, wt_hbm, mask_v, w_v, rows_v, sem_in, sem_out):
    wid = lax.axis_index("s") * NC + lax.axis_index("c")
    base = wid * R
    cp_m = pltpu.async_copy(mask_hbm.at[pl.ds(base, R)], mask_v, sem_in)
    cp_w = pltpu.async_copy(w_hbm.at[pl.ds(base, R)], w_v, sem_in)

    zeros = jnp.zeros((L,), jnp.float32)

    @pl.loop(0, R * D_IN // L, unroll=32)
    def _zero(i):
        rows_v[pl.ds(i * L, L)] = zeros

    cp_m.wait()
    cp_w.wait()

    CH = 2
    RC = R // CH
    cps = []
    for c in range(CH):
        @plsc.parallel_loop(c * RC, (c + 1) * RC)
        def _scatter(r):
            off = (r * D_IN).astype(jnp.int32)
            idx = mask_v[r, pl.ds(0, L)]
            val = w_v[r, pl.ds(0, L)]
            for j in range(K // L):
                nidx = nval = None
                if j + 1 < K // L:
                    nidx = mask_v[r, pl.ds((j + 1) * L, L)]
                    nval = w_v[r, pl.ds((j + 1) * L, L)]
                plsc.addupdate_scatter(rows_v, [idx + off], val)
                if nidx is not None:
                    idx, val = nidx, nval

        cps.append(pltpu.async_copy(
            rows_v.at[pl.ds(c * RC * D_IN, RC * D_IN)],
            wt_hbm.at[pl.ds((base + c * RC) * D_IN, RC * D_IN)],
            sem_out,
        ))
    for cp in cps:
        cp.wait()


BO = 512


def _mm_body(x_ref, wt_ref, b_ref, o_ref):
    wtb = wt_ref[...].reshape(BO, D_IN)
    o_ref[...] = (
        lax.dot_general(
            x_ref[...],
            wtb,
            dimension_numbers=(((1,), (1,)), ((), ())),
            preferred_element_type=jnp.float32,
            precision=lax.Precision.DEFAULT,
        )
        + b_ref[...]
    )


def _matmul(x, wt_flat, bias2d):
    return pl.pallas_call(
        _mm_body,
        grid=(D_OUT // BO,),
        in_specs=[
            pl.BlockSpec((N, D_IN), lambda i: (0, 0)),
            pl.BlockSpec((BO * D_IN,), lambda i: (i,)),
            pl.BlockSpec((1, BO), lambda i: (0, i)),
        ],
        out_specs=pl.BlockSpec((N, BO), lambda i: (0, i)),
        out_shape=jax.ShapeDtypeStruct((N, D_OUT), jnp.float32),
    )(x, wt_flat, bias2d)


def kernel(input, input_mask, condensed_weight, bias):
    wt = _densify(input_mask, condensed_weight)
    return _matmul(input, wt, bias.reshape(1, D_OUT))
